# Initial kernel scaffold; baseline (speedup 1.0000x reference)
import jax
import jax.numpy as jnp
from jax.experimental import pallas as pl


def kernel(xyz, params):
    raise NotImplementedError("write your pallas kernel here")


# speedup vs baseline: 10.9355x; 10.9355x over previous
import functools

import jax
import jax.numpy as jnp
from jax.experimental import pallas as pl

_F32 = jnp.float32
_NEG = -1.0e30


def _cumsum_lanes(x):
    n = x.shape[-1]
    sh = 1
    while sh < n:
        x = x + jnp.pad(x[:, :-sh], ((0, 0), (sh, 0)))
        sh *= 2
    return x


def _gather_lanes(src, idx):
    n = src.shape[1]
    nb = n // 128
    if nb == 1:
        return jnp.take_along_axis(src, idx, axis=1)
    land = jnp.bitwise_and(idx, 127)
    blk = jnp.right_shift(idx, 7)
    out = jnp.zeros(idx.shape, src.dtype)
    for b in range(nb):
        g = jnp.take_along_axis(src[:, b * 128:(b + 1) * 128], land, axis=1)
        out = jnp.where(blk == b, g, out)
    return out


def _gather_lanes_shared(src, idxf, rows):
    n = src.shape[1]
    m = idxf.shape[1]
    nb = n // 128
    if nb == 1:
        idxb = jnp.broadcast_to(idxf, (rows, m))
        return jnp.take_along_axis(src, idxb, axis=1)
    land = jnp.broadcast_to(jnp.bitwise_and(idxf, 127), (rows, m))
    blk = jnp.right_shift(idxf, 7)
    out = jnp.zeros((rows, m), src.dtype)
    for b in range(nb):
        g = jnp.take_along_axis(src[:, b * 128:(b + 1) * 128], land, axis=1)
        out = jnp.where(blk == b, g, out)
    return out


def _ball_first_k(c8, x8, r2, kk, kk_pad):
    s_blk = c8.shape[1]
    n = x8.shape[1]
    xx = jnp.sum(x8 * x8, axis=0, keepdims=True)
    cc = jnp.sum(c8 * c8, axis=0, keepdims=True)
    cc_col = jnp.transpose(cc)
    dotcx = jax.lax.dot_general(c8, x8, (((0,), (0,)), ((), ())),
                                preferred_element_type=_F32)
    d = (-2.0 * dotcx + cc_col) + xx
    maskf = (d <= jnp.float32(r2)).astype(_F32)
    cnt = _cumsum_lanes(maskf)
    total = cnt[:, n - 1:n]
    iota_k = jax.lax.broadcasted_iota(
        jnp.int32, (s_blk, kk_pad), 1).astype(_F32)
    tgt = iota_k + 1.0
    pos = jnp.full((s_blk, kk_pad), -1, jnp.int32)
    j = n // 2
    while j >= 1:
        cand = pos + j
        gv = _gather_lanes(cnt, cand)
        pos = jnp.where(gv < tgt, cand, pos)
        j //= 2
    idx = jnp.minimum(pos + 1, n - 1)
    idxf = jnp.transpose(idx).reshape(1, s_blk * kk_pad)
    iota_k0 = jax.lax.broadcasted_iota(
        jnp.int32, (kk_pad, s_blk), 0).astype(_F32)
    nvalid = jnp.maximum(jnp.minimum(jnp.transpose(total), jnp.float32(kk)),
                         1.0)
    validf = iota_k0 < nvalid
    return idxf, validf


def _smod_iota(s_blk, m):
    i = jax.lax.broadcasted_iota(jnp.int32, (1, m), 1)
    return jnp.bitwise_and(i, s_blk - 1)


def _mlp_max(h, layers, validf, s_blk, kk_pad, out_ref):
    for wt, b in layers:
        h = jnp.maximum(jnp.dot(wt, h, preferred_element_type=_F32) + b, 0.0)
    d3 = h.shape[0]
    h3 = h.reshape(d3, kk_pad, s_blk)
    h3 = jnp.where(validf[None], h3, _NEG)
    out_ref[0] = jnp.max(h3, axis=1)


def _fps_body(x_ref, o_ref, *, npoint):
    x = x_ref[...]
    bb, _, n = x.shape
    iota_n = jax.lax.broadcasted_iota(jnp.int32, (bb, n), 1)
    iota_s = jax.lax.broadcasted_iota(jnp.int32, (1, 1, npoint), 2)

    def body(i, st):
        dist, far, acc = st
        oh = (iota_n == far).astype(_F32)
        c = jnp.sum(x * oh[:, None, :], axis=2)
        d = jnp.sum((x - c[:, :, None]) ** 2, axis=1)
        dist = jnp.minimum(dist, d)
        m = jnp.max(dist, axis=1, keepdims=True)
        far = jnp.min(jnp.where(dist == m, iota_n, n), axis=1, keepdims=True)
        acc = acc + c[:, :, None] * (iota_s == i).astype(_F32)
        return dist, far, acc

    dist0 = jnp.full((bb, n), 1e10, _F32)
    far0 = jnp.zeros((bb, 1), jnp.int32)
    acc0 = jnp.zeros((bb, 3, npoint), _F32)
    _, _, acc = jax.lax.fori_loop(0, npoint, body, (dist0, far0, acc0))
    o_ref[...] = acc


def _fps_call(xyz3, npoint):
    bb = xyz3.shape[0]
    return pl.pallas_call(
        functools.partial(_fps_body, npoint=npoint),
        out_shape=jax.ShapeDtypeStruct((bb, 3, npoint), _F32),
    )(xyz3)


def _group_raw_body(xyzp_ref, new_ref, w1_ref, b1_ref, w2_ref, b2_ref,
                    w3_ref, b3_ref, o_ref, *, r2, kk, s_blk):
    x8 = xyzp_ref[0]
    c = new_ref[0]
    c8 = jnp.concatenate([c, jnp.zeros((5, s_blk), _F32)], axis=0)
    kkp = max(kk, 32)
    idxf, validf = _ball_first_k(c8, x8, r2, kk, kkp)
    g = _gather_lanes_shared(x8, idxf, 8)
    c_rep = _gather_lanes_shared(c8, _smod_iota(s_blk, s_blk * kkp), 8)
    h0 = g - c_rep
    h = jnp.maximum(
        jnp.dot(w1_ref[...], h0, preferred_element_type=_F32) + b1_ref[...],
        0.0)
    layers = ((w2_ref[...], b2_ref[...]), (w3_ref[...], b3_ref[...]))
    _mlp_max(h, layers, validf, s_blk, kkp, o_ref)


def _group_raw_call(xyzp, new3, ws, r2, kk, s_blk):
    bb, _, n = xyzp.shape
    s = new3.shape[2]
    (w1, b1), (w2, b2), (w3, b3) = ws
    d1, d2, d3 = w1.shape[1], w2.shape[1], w3.shape[1]
    w1t = jnp.transpose(jnp.concatenate(
        [w1, jnp.zeros((5, d1), _F32)], axis=0))
    w2t, w3t = jnp.transpose(w2), jnp.transpose(w3)
    args = (xyzp, new3, w1t, b1.reshape(d1, 1), w2t, b2.reshape(d2, 1),
            w3t, b3.reshape(d3, 1))
    full = lambda shp: pl.BlockSpec(shp, lambda b, si: (0,) * len(shp))
    return pl.pallas_call(
        functools.partial(_group_raw_body, r2=r2, kk=kk, s_blk=s_blk),
        grid=(bb, s // s_blk),
        in_specs=[
            pl.BlockSpec((1, 8, n), lambda b, si: (b, 0, 0)),
            pl.BlockSpec((1, 3, s_blk), lambda b, si: (b, 0, si)),
            full((d1, 8)), full((d1, 1)),
            full((d2, d1)), full((d2, 1)),
            full((d3, d2)), full((d3, 1)),
        ],
        out_specs=pl.BlockSpec((1, d3, s_blk), lambda b, si: (b, 0, si)),
        out_shape=jax.ShapeDtypeStruct((bb, d3, s), _F32),
    )(*args)


def _group_proj_body(xyzp_ref, pts_ref, new_ref, w1_ref, w1x_ref, b1_ref,
                     w2_ref, b2_ref, w3_ref, b3_ref, o_ref,
                     *, r2, kk, s_blk):
    x8 = xyzp_ref[0]
    c = new_ref[0]
    c8 = jnp.concatenate([c, jnp.zeros((5, s_blk), _F32)], axis=0)
    kkp = max(kk, 32)
    idxf, validf = _ball_first_k(c8, x8, r2, kk, kkp)
    ft = jnp.concatenate([pts_ref[0], x8], axis=0)
    d1 = w1_ref.shape[0]
    yt = jnp.dot(w1_ref[...], ft, preferred_element_type=_F32)
    g = _gather_lanes_shared(yt, idxf, d1)
    coff = jnp.dot(w1x_ref[...], c8, preferred_element_type=_F32)
    coff_rep = _gather_lanes_shared(coff, _smod_iota(s_blk, s_blk * kkp), d1)
    h = jnp.maximum(g - coff_rep + b1_ref[...], 0.0)
    layers = ((w2_ref[...], b2_ref[...]), (w3_ref[...], b3_ref[...]))
    _mlp_max(h, layers, validf, s_blk, kkp, o_ref)


def _group_proj_call(xyzp, pts, new3, ws, r2, kk, s_blk):
    bb, _, n = xyzp.shape
    cc = pts.shape[1]
    s = new3.shape[2]
    (w1, b1), (w2, b2), (w3, b3) = ws
    d1, d2, d3 = w1.shape[1], w2.shape[1], w3.shape[1]
    w1t = jnp.transpose(jnp.concatenate(
        [w1, jnp.zeros((5, d1), _F32)], axis=0))
    w1xt = jnp.transpose(jnp.concatenate(
        [w1[cc:cc + 3], jnp.zeros((5, d1), _F32)], axis=0))
    w2t, w3t = jnp.transpose(w2), jnp.transpose(w3)
    args = (xyzp, pts, new3, w1t, w1xt, b1.reshape(d1, 1), w2t,
            b2.reshape(d2, 1), w3t, b3.reshape(d3, 1))
    full = lambda shp: pl.BlockSpec(shp, lambda b, si: (0,) * len(shp))
    return pl.pallas_call(
        functools.partial(_group_proj_body, r2=r2, kk=kk, s_blk=s_blk),
        grid=(bb, s // s_blk),
        in_specs=[
            pl.BlockSpec((1, 8, n), lambda b, si: (b, 0, 0)),
            pl.BlockSpec((1, cc, n), lambda b, si: (b, 0, 0)),
            pl.BlockSpec((1, 3, s_blk), lambda b, si: (b, 0, si)),
            full((d1, cc + 8)), full((d1, 8)), full((d1, 1)),
            full((d2, d1)), full((d2, 1)),
            full((d3, d2)), full((d3, 1)),
        ],
        out_specs=pl.BlockSpec((1, d3, s_blk), lambda b, si: (b, 0, si)),
        out_shape=jax.ShapeDtypeStruct((bb, d3, s), _F32),
    )(*args)


def _inorm_cols(x, eps=1e-5):
    m = jnp.mean(x, axis=0, keepdims=True)
    v = jnp.mean((x - m) ** 2, axis=0, keepdims=True)
    return (x - m) / jnp.sqrt(v + eps)


def _sa3_head_body(f_ref, w1_ref, b1_ref, w2_ref, b2_ref, w3_ref, b3_ref,
                   wf1_ref, bf1_ref, wf2_ref, bf2_ref, wf3_ref, bf3_ref,
                   o_ref, *, bb, npt):
    h = f_ref[...]
    h = jnp.maximum(jnp.dot(w1_ref[...], h, preferred_element_type=_F32)
                    + b1_ref[...], 0.0)
    h = jnp.maximum(jnp.dot(w2_ref[...], h, preferred_element_type=_F32)
                    + b2_ref[...], 0.0)
    h = jnp.maximum(jnp.dot(w3_ref[...], h, preferred_element_type=_F32)
                    + b3_ref[...], 0.0)
    l3 = jnp.max(h.reshape(1024, bb, npt), axis=2)
    x = jnp.concatenate(
        [jnp.zeros((3, bb), _F32), l3, jnp.zeros((5, bb), _F32)], axis=0)
    o = _inorm_cols(jnp.dot(wf1_ref[...], x, preferred_element_type=_F32)
                    + bf1_ref[...])
    o = _inorm_cols(jnp.dot(wf2_ref[...], o, preferred_element_type=_F32)
                    + bf2_ref[...])
    o = _inorm_cols(jnp.dot(wf3_ref[...], o, preferred_element_type=_F32)
                    + bf3_ref[...])
    o_ref[...] = jnp.where(o >= 0.0, o, 0.01 * o)


def _sa3_head_call(f2, sa3, fc1, fc2, fc3, bb, npt):
    (w1, b1), (w2, b2), (w3, b3) = sa3
    wf1, bf1 = fc1
    wf2, bf2 = fc2
    wf3, bf3 = fc3
    w1t = jnp.transpose(jnp.concatenate(
        [w1, jnp.zeros((5, w1.shape[1]), _F32)], axis=0))
    wf1t = jnp.transpose(jnp.concatenate(
        [wf1, jnp.zeros((5, wf1.shape[1]), _F32)], axis=0))
    args = (f2, w1t, b1.reshape(-1, 1), jnp.transpose(w2), b2.reshape(-1, 1),
            jnp.transpose(w3), b3.reshape(-1, 1), wf1t, bf1.reshape(-1, 1),
            jnp.transpose(wf2), bf2.reshape(-1, 1), jnp.transpose(wf3),
            bf3.reshape(-1, 1))
    return pl.pallas_call(
        functools.partial(_sa3_head_body, bb=bb, npt=npt),
        out_shape=jax.ShapeDtypeStruct((256, bb), _F32),
    )(*args)


def kernel(xyz, params):
    bb, _, n = xyz.shape
    z5 = lambda nn: jnp.zeros((bb, 5, nn), _F32)
    new1 = _fps_call(xyz, 512)
    xyzp1 = jnp.concatenate([xyz, z5(n)], axis=1)
    outs1 = [
        _group_raw_call(xyzp1, new1, params["sa1"][i], r * r, k, 128)
        for i, (r, k) in enumerate(((0.1, 16), (0.2, 32), (0.4, 128)))
    ]
    l1_pts = jnp.concatenate(outs1, axis=1)
    new2 = _fps_call(new1, 128)
    xyzp2 = jnp.concatenate([new1, z5(512)], axis=1)
    outs2 = [
        _group_proj_call(xyzp2, l1_pts, new2, params["sa2"][i], r * r, k, 128)
        for i, (r, k) in enumerate(((0.2, 32), (0.4, 64), (0.8, 128)))
    ]
    l2_pts = jnp.concatenate(outs2, axis=1)
    f = jnp.concatenate([new2, l2_pts, z5(128)], axis=1)
    f2 = jnp.transpose(f, (1, 0, 2)).reshape(648, bb * 128)
    o = _sa3_head_call(f2, params["sa3"], params["fc1"], params["fc2"],
                       params["fc3"], bb, 128)
    return jnp.transpose(o)
